# baseline (speedup 1.0000x reference)
import functools

import jax
import jax.numpy as jnp
from jax import lax
from jax.experimental import pallas as pl
from jax.experimental.pallas import tpu as pltpu
from jax.experimental.pallas import tpu_sc as plsc

_NC = 2
_NS = 16
_NW = _NC * _NS
_CHUNK = 128

_BN = 1024
_ZREP = 489
_DREP = 123


def _mxu_t(a_ref, eye_ref):
    return lax.dot_general(a_ref[...], eye_ref[...], (((0,), (0,)), ((), ())),
                           preferred_element_type=jnp.float32)


def _prep_z_body(zd, in_ref, eye_ref, out_ref):
    h = pl.program_id(1)
    t = _mxu_t(in_ref, eye_ref)
    for hh in range(2):
        @pl.when(h == hh)
        def _():
            out_ref[:, hh * zd:(hh + 1) * zd] = t


def _prep_d_body(ed, in_ref, eye_ref, out_ref):
    k = pl.program_id(1)
    t = _mxu_t(in_ref, eye_ref)
    for kk in range(8):
        @pl.when(k == kk)
        def _():
            out_ref[:, kk * ed:(kk + 1) * ed] = t


def _tc_prep(zt, dt):
    zd = zt.shape[0]
    ed = dt.shape[0]
    nbins = zt.shape[1]
    last = (nbins - 1) // _BN

    def _zmap(k, i):
        return (0, jnp.minimum(i + k * _ZREP, last))

    def _dmap(k, i):
        return (0, jnp.minimum(i + k * _DREP, last))

    z2 = pl.pallas_call(
        functools.partial(_prep_z_body, zd),
        grid=(_ZREP, 2),
        in_specs=[pl.BlockSpec((zd, _BN), lambda i, h: _zmap(h, i)),
                  pl.BlockSpec((zd, zd), lambda i, h: (0, 0))],
        out_specs=pl.BlockSpec((_BN, 2 * zd), lambda i, h: (i, 0)),
        out_shape=jax.ShapeDtypeStruct((_ZREP * _BN, 2 * zd), jnp.float32),
    )(zt, jnp.eye(zd, dtype=jnp.float32))
    d2 = pl.pallas_call(
        functools.partial(_prep_d_body, ed),
        grid=(_DREP, 8),
        in_specs=[pl.BlockSpec((ed, _BN), lambda i, k: _dmap(k, i)),
                  pl.BlockSpec((ed, ed), lambda i, k: (0, 0))],
        out_specs=pl.BlockSpec((_BN, 8 * ed), lambda i, k: (i, 0)),
        out_shape=jax.ShapeDtypeStruct((_DREP * _BN, 8 * ed), jnp.float32),
    )(dt, jnp.eye(ed, dtype=jnp.float32))
    return z2, d2


def _sc_gather_body(nchunks, bpw, idz_hbm, idd_hbm, z_hbm, d_hbm,
                    zout_hbm, dout_hbm, idz_v, idd_v, rows, sem):
    wid = lax.axis_index("s") * _NC + lax.axis_index("c")
    base = wid * bpw
    pltpu.sync_copy(idz_hbm.at[pl.ds(base, bpw)], idz_v)
    pltpu.sync_copy(idd_hbm.at[pl.ds(base, bpw)], idd_v)
    copies = []
    for j in range(nchunks):
        sl = pl.ds(j * _CHUNK, _CHUNK)
        copies.append(pltpu.async_copy(z_hbm.at[idz_v.at[sl]], rows.at[sl], sem))
    for cp in copies:
        cp.wait()
    pltpu.sync_copy(rows, zout_hbm.at[pl.ds(base, bpw)])
    copies = []
    for j in range(nchunks):
        sl = pl.ds(j * _CHUNK, _CHUNK)
        copies.append(pltpu.async_copy(d_hbm.at[idd_v.at[sl]], rows.at[sl], sem))
    for cp in copies:
        cp.wait()
    pltpu.sync_copy(rows, dout_hbm.at[pl.ds(base, bpw)])


def _sc_gather(idz, idd, z2, d2):
    b = idz.shape[0]
    bpw = b // _NW
    nchunks = bpw // _CHUNK
    mesh = plsc.VectorSubcoreMesh(core_axis_name="c", subcore_axis_name="s")
    run = pl.kernel(
        functools.partial(_sc_gather_body, nchunks, bpw),
        out_type=(
            jax.ShapeDtypeStruct((b, 128), jnp.float32),
            jax.ShapeDtypeStruct((b, 128), jnp.float32),
        ),
        mesh=mesh,
        scratch_types=[
            pltpu.VMEM((bpw,), jnp.int32),
            pltpu.VMEM((bpw,), jnp.int32),
            pltpu.VMEM((bpw, 128), jnp.float32),
            pltpu.SemaphoreType.DMA,
        ],
        compiler_params=pltpu.CompilerParams(use_tc_tiling_on_sc=True),
    )
    return run(idz, idd, z2, d2)


def _tc_mlp_body(xd, zd, ed, x_ref, zh_ref, ds_ref, zbuf_ref, dbuf_ref,
                 w1x_ref, w1z_ref, b1_ref, w2_ref, b2_ref, wf_ref, out_ref):
    zhalf = zh_ref[0]
    dslot = ds_ref[0]
    zsel = jnp.where(zhalf == 0, zbuf_ref[:, :zd], zbuf_ref[:, zd:])
    dsel = jnp.zeros((dbuf_ref.shape[0], ed), jnp.float32)
    for k in range(8):
        dsel += jnp.where(dslot == k, dbuf_ref[:, k * ed:(k + 1) * ed], 0.0)
    h = x_ref[...] @ w1x_ref[...] + zsel @ w1z_ref[...] + b1_ref[...]
    h = jnp.maximum(h, 0.0)
    intrinsic = h @ w2_ref[...] + b2_ref[...]
    gated = jax.nn.sigmoid(dsel)
    out_ref[...] = jnp.sum(intrinsic * gated * wf_ref[...], axis=1)[None, None, :]


def _tc_mlp(x, zhalf, dslot, zbuf, dbuf, W1, b1, W2, b2, Wf):
    b, xd = x.shape
    zd = W1.shape[0] - xd
    ed = W2.shape[1]
    hd = W1.shape[1]
    bm = 2048
    grid = b // bm
    w1x = W1[:xd]
    w1z = W1[xd:]
    zh3 = zhalf.reshape(grid, bm, 1)
    ds3 = dslot.reshape(grid, bm, 1)
    out = pl.pallas_call(
        functools.partial(_tc_mlp_body, xd, zd, ed),
        grid=(grid,),
        in_specs=[
            pl.BlockSpec((bm, xd), lambda i: (i, 0)),
            pl.BlockSpec((1, bm, 1), lambda i: (i, 0, 0)),
            pl.BlockSpec((1, bm, 1), lambda i: (i, 0, 0)),
            pl.BlockSpec((bm, 128), lambda i: (i, 0)),
            pl.BlockSpec((bm, 128), lambda i: (i, 0)),
            pl.BlockSpec((xd, hd), lambda i: (0, 0)),
            pl.BlockSpec((zd, hd), lambda i: (0, 0)),
            pl.BlockSpec((1, hd), lambda i: (0, 0)),
            pl.BlockSpec((hd, ed), lambda i: (0, 0)),
            pl.BlockSpec((1, ed), lambda i: (0, 0)),
            pl.BlockSpec((1, ed), lambda i: (0, 0)),
        ],
        out_specs=pl.BlockSpec((1, 1, bm), lambda i: (i, 0, 0)),
        out_shape=jax.ShapeDtypeStruct((grid, 1, bm), jnp.float32),
    )(x, zh3, ds3, zbuf, dbuf, w1x, w1z, b1.reshape(1, hd), W2,
      b2.reshape(1, ed), Wf.reshape(1, ed))
    return out.reshape(b)


def kernel(x, bin_ids, latent_z, latent_d, W1, b1, W2, b2, Wf):
    zrows = _ZREP * _BN
    drows = _DREP * _BN
    z2, d2 = _tc_prep(latent_z.T, latent_d.T)
    zhalf = (bin_ids >= zrows).astype(jnp.int32)
    idz = bin_ids - zhalf * zrows
    dslot = bin_ids // drows
    idd = bin_ids - dslot * drows
    zbuf, dbuf = _sc_gather(idz, idd, z2, d2)
    return _tc_mlp(x, zhalf, dslot, zbuf, dbuf, W1, b1, W2, b2, Wf)


# speedup vs baseline: 2.6371x; 2.6371x over previous
import functools

import jax
import jax.numpy as jnp
from jax import lax
from jax.experimental import pallas as pl
from jax.experimental.pallas import tpu as pltpu
from jax.experimental.pallas import tpu_sc as plsc

_NC = 2
_NS = 16
_NW = _NC * _NS
_CHUNK = 128


_BN = 4096
_ZREP = 123
_DREP = 31


def _mxu_t(a, eye):
    return lax.dot_general(a, eye, (((0,), (0,)), ((), ())),
                           preferred_element_type=jnp.float32)


def _prep_z_body(in0_ref, in1_ref, eye_ref, out_ref):
    eye = eye_ref[...]
    out_ref[...] = jnp.concatenate(
        [_mxu_t(in0_ref[...], eye), _mxu_t(in1_ref[...], eye)], axis=1)


def _prep_d_body(*refs):
    ins = refs[:-2]
    eye = refs[-2][...]
    out_ref = refs[-1]
    out_ref[...] = jnp.concatenate([_mxu_t(r[...], eye) for r in ins], axis=1)


def _tc_prep(zt, dt):
    zd = zt.shape[0]
    ed = dt.shape[0]
    last = (zt.shape[1] - 1) // _BN

    def _zmap(k, i):
        return (0, jnp.minimum(i + k * _ZREP, last))

    def _dmap(k, i):
        return (0, jnp.minimum(i + k * _DREP, last))

    z2 = pl.pallas_call(
        _prep_z_body,
        grid=(_ZREP,),
        in_specs=[pl.BlockSpec((zd, _BN), functools.partial(_zmap, 0)),
                  pl.BlockSpec((zd, _BN), functools.partial(_zmap, 1)),
                  pl.BlockSpec((zd, zd), lambda i: (0, 0))],
        out_specs=pl.BlockSpec((_BN, 2 * zd), lambda i: (i, 0)),
        out_shape=jax.ShapeDtypeStruct((_ZREP * _BN, 2 * zd), jnp.float32),
    )(zt, zt, jnp.eye(zd, dtype=jnp.float32))
    d2 = pl.pallas_call(
        _prep_d_body,
        grid=(_DREP,),
        in_specs=[pl.BlockSpec((ed, _BN), functools.partial(_dmap, k))
                  for k in range(8)] +
                 [pl.BlockSpec((ed, ed), lambda i: (0, 0))],
        out_specs=pl.BlockSpec((_BN, 8 * ed), lambda i: (i, 0)),
        out_shape=jax.ShapeDtypeStruct((_DREP * _BN, 8 * ed), jnp.float32),
    )(*([dt] * 8 + [jnp.eye(ed, dtype=jnp.float32)]))
    return z2, d2


def _sc_gather_body(nchunks, bpw, idz_hbm, idd_hbm, z_hbm, d_hbm,
                    zout_hbm, dout_hbm, idz_v, idd_v, rows, sem):
    wid = lax.axis_index("s") * _NC + lax.axis_index("c")
    base = wid * bpw
    pltpu.sync_copy(idz_hbm.at[pl.ds(base, bpw)], idz_v)
    pltpu.sync_copy(idd_hbm.at[pl.ds(base, bpw)], idd_v)
    copies = []
    for j in range(nchunks):
        sl = pl.ds(j * _CHUNK, _CHUNK)
        copies.append(pltpu.async_copy(z_hbm.at[idz_v.at[sl]], rows.at[sl], sem))
    for cp in copies:
        cp.wait()
    pltpu.sync_copy(rows, zout_hbm.at[pl.ds(base, bpw)])
    copies = []
    for j in range(nchunks):
        sl = pl.ds(j * _CHUNK, _CHUNK)
        copies.append(pltpu.async_copy(d_hbm.at[idd_v.at[sl]], rows.at[sl], sem))
    for cp in copies:
        cp.wait()
    pltpu.sync_copy(rows, dout_hbm.at[pl.ds(base, bpw)])


def _sc_gather(idz, idd, z2, d2):
    b = idz.shape[0]
    bpw = b // _NW
    nchunks = bpw // _CHUNK
    mesh = plsc.VectorSubcoreMesh(core_axis_name="c", subcore_axis_name="s")
    run = pl.kernel(
        functools.partial(_sc_gather_body, nchunks, bpw),
        out_type=(
            jax.ShapeDtypeStruct((b, 128), jnp.float32),
            jax.ShapeDtypeStruct((b, 128), jnp.float32),
        ),
        mesh=mesh,
        scratch_types=[
            pltpu.VMEM((bpw,), jnp.int32),
            pltpu.VMEM((bpw,), jnp.int32),
            pltpu.VMEM((bpw, 128), jnp.float32),
            pltpu.SemaphoreType.DMA,
        ],
        compiler_params=pltpu.CompilerParams(use_tc_tiling_on_sc=True),
    )
    return run(idz, idd, z2, d2)


def _tc_mlp_body(xd, zd, ed, x_ref, zh_ref, ds_ref, zbuf_ref, dbuf_ref,
                 w1x_ref, w1z_ref, b1_ref, w2_ref, b2_ref, wf_ref, out_ref):
    zhalf = zh_ref[0]
    dslot = ds_ref[0]
    zsel = jnp.where(zhalf == 0, zbuf_ref[:, :zd], zbuf_ref[:, zd:])
    dsel = jnp.zeros((dbuf_ref.shape[0], ed), jnp.float32)
    for k in range(8):
        dsel += jnp.where(dslot == k, dbuf_ref[:, k * ed:(k + 1) * ed], 0.0)
    h = x_ref[...] @ w1x_ref[...] + zsel @ w1z_ref[...] + b1_ref[...]
    h = jnp.maximum(h, 0.0)
    intrinsic = h @ w2_ref[...] + b2_ref[...]
    gated = jax.nn.sigmoid(dsel)
    out_ref[...] = jnp.sum(intrinsic * gated * wf_ref[...], axis=1)[None, None, :]


def _tc_mlp(x, zhalf, dslot, zbuf, dbuf, W1, b1, W2, b2, Wf):
    b, xd = x.shape
    zd = W1.shape[0] - xd
    ed = W2.shape[1]
    hd = W1.shape[1]
    bm = 2048
    grid = b // bm
    w1x = W1[:xd]
    w1z = W1[xd:]
    zh3 = zhalf.reshape(grid, bm, 1)
    ds3 = dslot.reshape(grid, bm, 1)
    out = pl.pallas_call(
        functools.partial(_tc_mlp_body, xd, zd, ed),
        grid=(grid,),
        in_specs=[
            pl.BlockSpec((bm, xd), lambda i: (i, 0)),
            pl.BlockSpec((1, bm, 1), lambda i: (i, 0, 0)),
            pl.BlockSpec((1, bm, 1), lambda i: (i, 0, 0)),
            pl.BlockSpec((bm, 128), lambda i: (i, 0)),
            pl.BlockSpec((bm, 128), lambda i: (i, 0)),
            pl.BlockSpec((xd, hd), lambda i: (0, 0)),
            pl.BlockSpec((zd, hd), lambda i: (0, 0)),
            pl.BlockSpec((1, hd), lambda i: (0, 0)),
            pl.BlockSpec((hd, ed), lambda i: (0, 0)),
            pl.BlockSpec((1, ed), lambda i: (0, 0)),
            pl.BlockSpec((1, ed), lambda i: (0, 0)),
        ],
        out_specs=pl.BlockSpec((1, 1, bm), lambda i: (i, 0, 0)),
        out_shape=jax.ShapeDtypeStruct((grid, 1, bm), jnp.float32),
    )(x, zh3, ds3, zbuf, dbuf, w1x, w1z, b1.reshape(1, hd), W2,
      b2.reshape(1, ed), Wf.reshape(1, ed))
    return out.reshape(b)


def kernel(x, bin_ids, latent_z, latent_d, W1, b1, W2, b2, Wf):
    zrows = _ZREP * _BN
    drows = _DREP * _BN
    z2, d2 = _tc_prep(latent_z.T, latent_d.T)
    zhalf = (bin_ids >= zrows).astype(jnp.int32)
    idz = bin_ids - zhalf * zrows
    dslot = bin_ids // drows
    idd = bin_ids - dslot * drows
    zbuf, dbuf = _sc_gather(idz, idd, z2, d2)
    return _tc_mlp(x, zhalf, dslot, zbuf, dbuf, W1, b1, W2, b2, Wf)


# speedup vs baseline: 2.7602x; 1.0467x over previous
import functools

import jax
import jax.numpy as jnp
from jax import lax
from jax.experimental import pallas as pl
from jax.experimental.pallas import tpu as pltpu
from jax.experimental.pallas import tpu_sc as plsc

_NC = 2
_NS = 16
_NW = _NC * _NS
_CHUNK = 128


_BN = 8192
_ZREP = 62
_DREP = 16


def _mxu_t(a, eye):
    return lax.dot_general(a, eye, (((0,), (0,)), ((), ())),
                           preferred_element_type=jnp.float32)


def _prep_z_body(in0_ref, in1_ref, eye_ref, out_ref):
    eye = eye_ref[...]
    out_ref[...] = jnp.concatenate(
        [_mxu_t(in0_ref[...], eye), _mxu_t(in1_ref[...], eye)], axis=1)


def _prep_d_body(*refs):
    ins = refs[:-2]
    eye = refs[-2][...]
    out_ref = refs[-1]
    out_ref[...] = jnp.concatenate([_mxu_t(r[...], eye) for r in ins], axis=1)


def _tc_prep(zt, dt):
    zd = zt.shape[0]
    ed = dt.shape[0]
    last = (zt.shape[1] - 1) // _BN

    def _zmap(k, i):
        return (0, jnp.minimum(i + k * _ZREP, last))

    def _dmap(k, i):
        return (0, jnp.minimum(i + k * _DREP, last))

    z2 = pl.pallas_call(
        _prep_z_body,
        grid=(_ZREP,),
        in_specs=[pl.BlockSpec((zd, _BN), functools.partial(_zmap, 0)),
                  pl.BlockSpec((zd, _BN), functools.partial(_zmap, 1)),
                  pl.BlockSpec((zd, zd), lambda i: (0, 0))],
        out_specs=pl.BlockSpec((_BN, 2 * zd), lambda i: (i, 0)),
        out_shape=jax.ShapeDtypeStruct((_ZREP * _BN, 2 * zd), jnp.float32),
    )(zt, zt, jnp.eye(zd, dtype=jnp.float32))
    d2 = pl.pallas_call(
        _prep_d_body,
        grid=(_DREP,),
        in_specs=[pl.BlockSpec((ed, _BN), functools.partial(_dmap, k))
                  for k in range(8)] +
                 [pl.BlockSpec((ed, ed), lambda i: (0, 0))],
        out_specs=pl.BlockSpec((_BN, 8 * ed), lambda i: (i, 0)),
        out_shape=jax.ShapeDtypeStruct((_DREP * _BN, 8 * ed), jnp.float32),
    )(*([dt] * 8 + [jnp.eye(ed, dtype=jnp.float32)]))
    return z2, d2


def _sc_gather_body(nchunks, bpw, idz_hbm, idd_hbm, z_hbm, d_hbm,
                    zout_hbm, dout_hbm, idz_v, idd_v, rows, sem):
    wid = lax.axis_index("s") * _NC + lax.axis_index("c")
    base = wid * bpw
    pltpu.sync_copy(idz_hbm.at[pl.ds(base, bpw)], idz_v)
    pltpu.sync_copy(idd_hbm.at[pl.ds(base, bpw)], idd_v)
    copies = []
    for j in range(nchunks):
        sl = pl.ds(j * _CHUNK, _CHUNK)
        copies.append(pltpu.async_copy(z_hbm.at[idz_v.at[sl]], rows.at[sl], sem))
    for cp in copies:
        cp.wait()
    pltpu.sync_copy(rows, zout_hbm.at[pl.ds(base, bpw)])
    copies = []
    for j in range(nchunks):
        sl = pl.ds(j * _CHUNK, _CHUNK)
        copies.append(pltpu.async_copy(d_hbm.at[idd_v.at[sl]], rows.at[sl], sem))
    for cp in copies:
        cp.wait()
    pltpu.sync_copy(rows, dout_hbm.at[pl.ds(base, bpw)])


def _sc_gather(idz, idd, z2, d2):
    b = idz.shape[0]
    bpw = b // _NW
    nchunks = bpw // _CHUNK
    mesh = plsc.VectorSubcoreMesh(core_axis_name="c", subcore_axis_name="s")
    run = pl.kernel(
        functools.partial(_sc_gather_body, nchunks, bpw),
        out_type=(
            jax.ShapeDtypeStruct((b, 128), jnp.float32),
            jax.ShapeDtypeStruct((b, 128), jnp.float32),
        ),
        mesh=mesh,
        scratch_types=[
            pltpu.VMEM((bpw,), jnp.int32),
            pltpu.VMEM((bpw,), jnp.int32),
            pltpu.VMEM((bpw, 128), jnp.float32),
            pltpu.SemaphoreType.DMA,
        ],
        compiler_params=pltpu.CompilerParams(use_tc_tiling_on_sc=True),
    )
    return run(idz, idd, z2, d2)


def _tc_mlp_body(xd, zd, ed, x_ref, zh_ref, ds_ref, zbuf_ref, dbuf_ref,
                 w1x_ref, w1z_ref, b1_ref, w2_ref, b2_ref, wf_ref, out_ref):
    zhalf = zh_ref[0]
    dslot = ds_ref[0]
    zsel = jnp.where(zhalf == 0, zbuf_ref[:, :zd], zbuf_ref[:, zd:])
    dsel = jnp.zeros((dbuf_ref.shape[0], ed), jnp.float32)
    for k in range(8):
        dsel += jnp.where(dslot == k, dbuf_ref[:, k * ed:(k + 1) * ed], 0.0)
    h = x_ref[...] @ w1x_ref[...] + zsel @ w1z_ref[...] + b1_ref[...]
    h = jnp.maximum(h, 0.0)
    intrinsic = h @ w2_ref[...] + b2_ref[...]
    gated = jax.nn.sigmoid(dsel)
    out_ref[...] = jnp.sum(intrinsic * gated * wf_ref[...], axis=1)[None, None, :]


def _tc_mlp(x, zhalf, dslot, zbuf, dbuf, W1, b1, W2, b2, Wf):
    b, xd = x.shape
    zd = W1.shape[0] - xd
    ed = W2.shape[1]
    hd = W1.shape[1]
    bm = 2048
    grid = b // bm
    w1x = W1[:xd]
    w1z = W1[xd:]
    zh3 = zhalf.reshape(grid, bm, 1)
    ds3 = dslot.reshape(grid, bm, 1)
    out = pl.pallas_call(
        functools.partial(_tc_mlp_body, xd, zd, ed),
        grid=(grid,),
        in_specs=[
            pl.BlockSpec((bm, xd), lambda i: (i, 0)),
            pl.BlockSpec((1, bm, 1), lambda i: (i, 0, 0)),
            pl.BlockSpec((1, bm, 1), lambda i: (i, 0, 0)),
            pl.BlockSpec((bm, 128), lambda i: (i, 0)),
            pl.BlockSpec((bm, 128), lambda i: (i, 0)),
            pl.BlockSpec((xd, hd), lambda i: (0, 0)),
            pl.BlockSpec((zd, hd), lambda i: (0, 0)),
            pl.BlockSpec((1, hd), lambda i: (0, 0)),
            pl.BlockSpec((hd, ed), lambda i: (0, 0)),
            pl.BlockSpec((1, ed), lambda i: (0, 0)),
            pl.BlockSpec((1, ed), lambda i: (0, 0)),
        ],
        out_specs=pl.BlockSpec((1, 1, bm), lambda i: (i, 0, 0)),
        out_shape=jax.ShapeDtypeStruct((grid, 1, bm), jnp.float32),
    )(x, zh3, ds3, zbuf, dbuf, w1x, w1z, b1.reshape(1, hd), W2,
      b2.reshape(1, ed), Wf.reshape(1, ed))
    return out.reshape(b)


def kernel(x, bin_ids, latent_z, latent_d, W1, b1, W2, b2, Wf):
    zrows = _ZREP * _BN
    drows = _DREP * _BN
    z2, d2 = _tc_prep(latent_z.T, latent_d.T)
    zhalf = (bin_ids >= zrows).astype(jnp.int32)
    idz = bin_ids - zhalf * zrows
    dslot = bin_ids // drows
    idd = bin_ids - dslot * drows
    zbuf, dbuf = _sc_gather(idz, idd, z2, d2)
    return _tc_mlp(x, zhalf, dslot, zbuf, dbuf, W1, b1, W2, b2, Wf)
